# baseline (speedup 1.0000x reference)
import functools

import jax
import jax.numpy as jnp
from jax import lax
from jax.experimental import pallas as pl
from jax.experimental.pallas import tpu as pltpu
from jax.experimental.pallas import tpu_sc as plsc

N_NODES = 10000
N_EDGES = 320000
NC = 2
NS = 16
NW = NC * NS
CHUNK = 128
N_CHUNKS = N_EDGES // CHUNK
ITERS = -(-N_CHUNKS // NW)
ROWS_PER_TILE = (N_NODES // NS) // 8 * 8
TAIL_BASE = ROWS_PER_TILE * NS
TAIL = N_NODES - TAIL_BASE
CPT = 80
E_PAD = NW * CPT * CHUNK
AGG_ROWS = N_NODES + 16
NBUF = 2
PHASES = 2
CPH = CPT // PHASES

ROW_BLK = 1000
GRID = N_NODES // ROW_BLK


def _seg_sum_partials(support, src, dst, zeros, d):
    mesh = plsc.VectorSubcoreMesh(
        core_axis_name="c", subcore_axis_name="s", num_cores=NC, num_subcores=NS
    )

    @functools.partial(
        pl.kernel,
        out_type=jax.ShapeDtypeStruct((NC, N_NODES, d), jnp.float32),
        mesh=mesh,
        scratch_types=[
            pltpu.VMEM((CHUNK,), jnp.int32),
            pltpu.VMEM((CHUNK,), jnp.int32),
            [pltpu.VMEM((CHUNK, d), jnp.float32) for _ in range(NBUF)],
            pltpu.VMEM_SHARED((AGG_ROWS, d), jnp.float32),
            [pltpu.SemaphoreType.DMA for _ in range(NBUF)],
            [pltpu.SemaphoreType.DMA for _ in range(NBUF)],
        ],
    )
    def k(support_hbm, src_hbm, dst_hbm, zeros_hbm, out_hbm,
          src_idx, dst_idx, rows, agg_sh, gsem, ssem):
        cid = lax.axis_index("c")
        sid = lax.axis_index("s")
        wid = sid * NC + cid

        pltpu.sync_copy(zeros_hbm, agg_sh.at[pl.ds(sid * ROWS_PER_TILE, ROWS_PER_TILE)])

        @pl.when(sid == 0)
        def _():
            pltpu.sync_copy(
                zeros_hbm.at[pl.ds(0, AGG_ROWS - TAIL_BASE)],
                agg_sh.at[pl.ds(TAIL_BASE, AGG_ROWS - TAIL_BASE)],
            )

        plsc.subcore_barrier()

        def gather_start(i, b):
            pltpu.async_copy(support_hbm.at[src_idx.at[i]], rows[b], gsem[b])

        def gather_wait(i, b):
            pltpu.make_async_copy(support_hbm.at[src_idx.at[i]], rows[b], gsem[b]).wait()

        def scatter_start(i, b):
            pltpu.async_copy(rows[b], agg_sh.at[dst_idx.at[i]], ssem[b], add=True)

        def scatter_wait(i, b):
            pltpu.make_async_copy(rows[b], agg_sh.at[dst_idx.at[i]], ssem[b]).wait()

        def body(i, carry):
            row = wid * CPT + i
            pltpu.sync_copy(src_hbm.at[row], src_idx)
            pltpu.sync_copy(dst_hbm.at[row], dst_idx)
            pltpu.sync_copy(support_hbm.at[src_idx], rows[0])
            pltpu.sync_copy(rows[0], agg_sh.at[dst_idx], add=True)
            return carry

        lax.fori_loop(0, CPT, body, None)

        plsc.subcore_barrier()
        pltpu.sync_copy(
            agg_sh.at[pl.ds(sid * ROWS_PER_TILE, ROWS_PER_TILE)],
            out_hbm.at[cid, pl.ds(sid * ROWS_PER_TILE, ROWS_PER_TILE)],
        )

        @pl.when(sid == 0)
        def _():
            pltpu.sync_copy(
                agg_sh.at[pl.ds(TAIL_BASE, TAIL)],
                out_hbm.at[cid, pl.ds(TAIL_BASE, TAIL)],
            )

    return k(support, src, dst, zeros)


def _mm_bias_relu(parts, w, b, n_out):
    def body(p_ref, w_ref, b_ref, o_ref):
        agg = p_ref[0] + p_ref[1]
        o_ref[...] = jnp.maximum(
            jnp.dot(agg, w_ref[...], preferred_element_type=jnp.float32) + b_ref[...],
            0.0,
        )

    d = parts.shape[2]
    return pl.pallas_call(
        body,
        grid=(GRID,),
        in_specs=[
            pl.BlockSpec((NC, ROW_BLK, d), lambda i: (0, i, 0)),
            pl.BlockSpec(w.shape, lambda i: (0, 0)),
            pl.BlockSpec((1, n_out), lambda i: (0, 0)),
        ],
        out_specs=pl.BlockSpec((ROW_BLK, n_out), lambda i: (i, 0)),
        out_shape=jax.ShapeDtypeStruct((N_NODES, n_out), jnp.float32),
    )(parts, w, b.reshape(1, n_out))


def _mm_bias_log_softmax(parts, w, b, n_out):
    def body(p_ref, w_ref, b_ref, o_ref):
        agg = p_ref[0] + p_ref[1]
        o = jnp.dot(agg, w_ref[...], preferred_element_type=jnp.float32) + b_ref[...]
        m = jnp.max(o, axis=1, keepdims=True)
        e = jnp.exp(o - m)
        s = jnp.sum(e, axis=1, keepdims=True)
        o_ref[...] = o - m - jnp.log(s)

    d = parts.shape[2]
    return pl.pallas_call(
        body,
        grid=(GRID,),
        in_specs=[
            pl.BlockSpec((NC, ROW_BLK, d), lambda i: (0, i, 0)),
            pl.BlockSpec(w.shape, lambda i: (0, 0)),
            pl.BlockSpec((1, n_out), lambda i: (0, 0)),
        ],
        out_specs=pl.BlockSpec((ROW_BLK, n_out), lambda i: (i, 0)),
        out_shape=jax.ShapeDtypeStruct((N_NODES, n_out), jnp.float32),
    )(parts, w, b.reshape(1, n_out))


def kernel(x, adjs, W0, b0, W1, b1):
    pad = E_PAD - N_EDGES
    src = jnp.concatenate(
        [adjs[0].astype(jnp.int32), jnp.zeros((pad,), jnp.int32)]
    ).reshape(NW * CPT, CHUNK)
    dst = jnp.concatenate(
        [adjs[1].astype(jnp.int32), jnp.full((pad,), N_NODES, jnp.int32)]
    ).reshape(NW * CPT, CHUNK)
    nfeat = x.shape[1]
    nhid = W0.shape[1]
    ncls = W1.shape[1]
    z = jnp.zeros((ROWS_PER_TILE, nfeat), jnp.float32)

    parts1 = _seg_sum_partials(x, src, dst, z, nfeat)
    h = _mm_bias_relu(parts1, W0, b0, nhid)
    parts2 = _seg_sum_partials(h, src, dst, z, nhid)
    return _mm_bias_log_softmax(parts2, W1, b1, ncls)


# speedup vs baseline: 2.1419x; 2.1419x over previous
import functools

import jax
import jax.numpy as jnp
from jax import lax
from jax.experimental import pallas as pl
from jax.experimental.pallas import tpu as pltpu
from jax.experimental.pallas import tpu_sc as plsc

N_NODES = 10000
N_EDGES = 320000
NC = 2
NS = 16
NW = NC * NS
CHUNK = 128
N_CHUNKS = N_EDGES // CHUNK
ITERS = -(-N_CHUNKS // NW)
ROWS_PER_TILE = (N_NODES // NS) // 8 * 8
TAIL_BASE = ROWS_PER_TILE * NS
TAIL = N_NODES - TAIL_BASE
CPT = 80
E_PAD = NW * CPT * CHUNK
AGG_ROWS = N_NODES + 16
NBUF = 2
PHASES = 2
CPH = CPT // PHASES

ROW_BLK = 1000
GRID = N_NODES // ROW_BLK


def _seg_sum_partials(support, src, dst, zeros, d):
    mesh = plsc.VectorSubcoreMesh(
        core_axis_name="c", subcore_axis_name="s", num_cores=NC, num_subcores=NS
    )

    @functools.partial(
        pl.kernel,
        out_type=jax.ShapeDtypeStruct((NC, N_NODES, d), jnp.float32),
        mesh=mesh,
        scratch_types=[
            pltpu.VMEM((CHUNK,), jnp.int32),
            pltpu.VMEM((CHUNK,), jnp.int32),
            [pltpu.VMEM((CHUNK, d), jnp.float32) for _ in range(NBUF)],
            pltpu.VMEM_SHARED((AGG_ROWS, d), jnp.float32),
            [pltpu.SemaphoreType.DMA for _ in range(NBUF)],
            [pltpu.SemaphoreType.DMA for _ in range(NBUF)],
        ],
    )
    def k(support_hbm, src_hbm, dst_hbm, zeros_hbm, out_hbm,
          src_idx, dst_idx, rows, agg_sh, gsem, ssem):
        cid = lax.axis_index("c")
        sid = lax.axis_index("s")
        wid = sid * NC + cid

        pltpu.sync_copy(zeros_hbm, agg_sh.at[pl.ds(sid * ROWS_PER_TILE, ROWS_PER_TILE)])

        @pl.when(sid == 0)
        def _():
            pltpu.sync_copy(
                zeros_hbm.at[pl.ds(0, AGG_ROWS - TAIL_BASE)],
                agg_sh.at[pl.ds(TAIL_BASE, AGG_ROWS - TAIL_BASE)],
            )

        plsc.subcore_barrier()

        def gather_start(i, b):
            pltpu.async_copy(support_hbm.at[src_idx.at[i]], rows[b], gsem[b])

        def gather_wait(i, b):
            pltpu.make_async_copy(support_hbm.at[src_idx.at[i]], rows[b], gsem[b]).wait()

        def scatter_start(i, b):
            pltpu.async_copy(rows[b], agg_sh.at[dst_idx.at[i]], ssem[b], add=True)

        def scatter_wait(i, b):
            pltpu.make_async_copy(rows[b], agg_sh.at[dst_idx.at[i]], ssem[b]).wait()

        start = 78 * wid + jnp.minimum(wid, 4)
        cnt = jnp.where(wid < 4, 79, 78)

        def body(i, carry):
            base = (start + i) * CHUNK
            pltpu.sync_copy(src_hbm.at[pl.ds(base, CHUNK)], src_idx)
            pltpu.sync_copy(dst_hbm.at[pl.ds(base, CHUNK)], dst_idx)
            pltpu.sync_copy(support_hbm.at[src_idx], rows[0])
            pltpu.sync_copy(rows[0], agg_sh.at[dst_idx], add=True)
            return carry

        lax.fori_loop(0, cnt, body, None)

        plsc.subcore_barrier()
        pltpu.sync_copy(
            agg_sh.at[pl.ds(sid * ROWS_PER_TILE, ROWS_PER_TILE)],
            out_hbm.at[cid, pl.ds(sid * ROWS_PER_TILE, ROWS_PER_TILE)],
        )

        @pl.when(sid == 0)
        def _():
            pltpu.sync_copy(
                agg_sh.at[pl.ds(TAIL_BASE, TAIL)],
                out_hbm.at[cid, pl.ds(TAIL_BASE, TAIL)],
            )

    return k(support, src, dst, zeros)


def _mm_bias_relu(parts, w, b, n_out):
    def body(p_ref, w_ref, b_ref, o_ref):
        agg = p_ref[0] + p_ref[1]
        o_ref[...] = jnp.maximum(
            jnp.dot(agg, w_ref[...], preferred_element_type=jnp.float32) + b_ref[...],
            0.0,
        )

    d = parts.shape[2]
    return pl.pallas_call(
        body,
        grid=(GRID,),
        in_specs=[
            pl.BlockSpec((NC, ROW_BLK, d), lambda i: (0, i, 0)),
            pl.BlockSpec(w.shape, lambda i: (0, 0)),
            pl.BlockSpec((1, n_out), lambda i: (0, 0)),
        ],
        out_specs=pl.BlockSpec((ROW_BLK, n_out), lambda i: (i, 0)),
        out_shape=jax.ShapeDtypeStruct((N_NODES, n_out), jnp.float32),
    )(parts, w, b.reshape(1, n_out))


def _mm_bias_log_softmax(parts, w, b, n_out):
    def body(p_ref, w_ref, b_ref, o_ref):
        agg = p_ref[0] + p_ref[1]
        o = jnp.dot(agg, w_ref[...], preferred_element_type=jnp.float32) + b_ref[...]
        m = jnp.max(o, axis=1, keepdims=True)
        e = jnp.exp(o - m)
        s = jnp.sum(e, axis=1, keepdims=True)
        o_ref[...] = o - m - jnp.log(s)

    d = parts.shape[2]
    return pl.pallas_call(
        body,
        grid=(GRID,),
        in_specs=[
            pl.BlockSpec((NC, ROW_BLK, d), lambda i: (0, i, 0)),
            pl.BlockSpec(w.shape, lambda i: (0, 0)),
            pl.BlockSpec((1, n_out), lambda i: (0, 0)),
        ],
        out_specs=pl.BlockSpec((ROW_BLK, n_out), lambda i: (i, 0)),
        out_shape=jax.ShapeDtypeStruct((N_NODES, n_out), jnp.float32),
    )(parts, w, b.reshape(1, n_out))


def kernel(x, adjs, W0, b0, W1, b1):
    src = adjs[0].astype(jnp.int32)
    dst = adjs[1].astype(jnp.int32)
    nfeat = x.shape[1]
    nhid = W0.shape[1]
    ncls = W1.shape[1]
    z = jnp.zeros((ROWS_PER_TILE, nfeat), jnp.float32)

    parts1 = _seg_sum_partials(x, src, dst, z, nfeat)
    h = _mm_bias_relu(parts1, W0, b0, nhid)
    parts2 = _seg_sum_partials(h, src, dst, z, nhid)
    return _mm_bias_log_softmax(parts2, W1, b1, ncls)
